# baseline (speedup 1.0000x reference)
import jax
import jax.numpy as jnp
from jax.experimental import pallas as pl
from jax.experimental.pallas import tpu as pltpu

R = 128
C = 32768
BC = 4096
NB = C // BC


def _body(x_ref, out_ref, run_max, run_idx, col_scratch):
    t = pl.program_id(0)

    @pl.when(t == 0)
    def _():
        col_scratch[...] = jax.lax.broadcasted_iota(jnp.int32, (R, BC), 1)

    @pl.when(t < NB)
    def _():
        x = x_ref[...]
        m = jnp.max(x, axis=1, keepdims=True)
        loc = jnp.argmax(x, axis=1).astype(jnp.int32).reshape(R, 1) + t * BC

        @pl.when(t == 0)
        def _():
            run_max[...] = m
            run_idx[...] = loc

        @pl.when(t > 0)
        def _():
            better = m > run_max[...]
            run_idx[...] = jnp.where(better, loc, run_idx[...])
            run_max[...] = jnp.maximum(m, run_max[...])

    @pl.when(t >= NB)
    def _():
        j = t - NB
        idx_s = run_idx[...] - j * BC
        out_ref[...] = (col_scratch[...] == idx_s).astype(jnp.float32)


@jax.jit
def kernel(Xsoft, P):
    del P
    return pl.pallas_call(
        _body,
        grid=(2 * NB,),
        in_specs=[
            pl.BlockSpec((R, BC), lambda t: (0, jnp.minimum(t, NB - 1))),
        ],
        out_specs=pl.BlockSpec(
            (R, BC), lambda t: (0, jnp.where(t < NB, 0, t - NB))
        ),
        out_shape=jax.ShapeDtypeStruct((R, C), jnp.float32),
        scratch_shapes=[
            pltpu.VMEM((R, 1), jnp.float32),
            pltpu.VMEM((R, 1), jnp.int32),
            pltpu.VMEM((R, BC), jnp.int32),
        ],
    )(Xsoft)


# speedup vs baseline: 1.0996x; 1.0996x over previous
import jax
import jax.numpy as jnp
from jax.experimental import pallas as pl
from jax.experimental.pallas import tpu as pltpu

R = 128
C = 32768
BC = 16384
NB = C // BC


def _body(x_ref, out_ref, run_max, run_idx, col_scratch):
    t = pl.program_id(0)

    @pl.when(t == 0)
    def _():
        col_scratch[...] = jax.lax.broadcasted_iota(jnp.int32, (R, BC), 1)

    @pl.when(t < NB)
    def _():
        x = x_ref[...]
        m = jnp.max(x, axis=1, keepdims=True)
        loc = jnp.argmax(x, axis=1).astype(jnp.int32).reshape(R, 1) + t * BC

        @pl.when(t == 0)
        def _():
            run_max[...] = m
            run_idx[...] = loc

        @pl.when(t > 0)
        def _():
            better = m > run_max[...]
            run_idx[...] = jnp.where(better, loc, run_idx[...])
            run_max[...] = jnp.maximum(m, run_max[...])

    @pl.when(t >= NB)
    def _():
        j = t - NB
        idx_s = run_idx[...] - j * BC
        out_ref[...] = (col_scratch[...] == idx_s).astype(jnp.float32)


@jax.jit
def kernel(Xsoft, P):
    del P
    return pl.pallas_call(
        _body,
        grid=(2 * NB,),
        in_specs=[
            pl.BlockSpec((R, BC), lambda t: (0, jnp.minimum(t, NB - 1))),
        ],
        out_specs=pl.BlockSpec(
            (R, BC), lambda t: (0, jnp.where(t < NB, 0, t - NB))
        ),
        out_shape=jax.ShapeDtypeStruct((R, C), jnp.float32),
        scratch_shapes=[
            pltpu.VMEM((R, 1), jnp.float32),
            pltpu.VMEM((R, 1), jnp.int32),
            pltpu.VMEM((R, BC), jnp.int32),
        ],
    )(Xsoft)


# speedup vs baseline: 1.2678x; 1.1530x over previous
import jax
import jax.numpy as jnp
from jax.experimental import pallas as pl
from jax.experimental.pallas import tpu as pltpu

R = 128
C = 32768
BC = 8192
NB = C // BC


def _body(x_ref, out_ref, run_max, run_idx, col_scratch):
    t = pl.program_id(0)

    @pl.when(t == 0)
    def _():
        col_scratch[...] = jax.lax.broadcasted_iota(jnp.int32, (R, BC), 1)

    @pl.when(t < NB)
    def _():
        x = x_ref[...]
        m = jnp.max(x, axis=1, keepdims=True)
        loc = jnp.argmax(x, axis=1).astype(jnp.int32).reshape(R, 1) + t * BC

        @pl.when(t == 0)
        def _():
            run_max[...] = m
            run_idx[...] = loc

        @pl.when(t > 0)
        def _():
            better = m > run_max[...]
            run_idx[...] = jnp.where(better, loc, run_idx[...])
            run_max[...] = jnp.maximum(m, run_max[...])

    @pl.when(t >= NB)
    def _():
        j = t - NB
        idx_s = run_idx[...] - j * BC
        out_ref[...] = (col_scratch[...] == idx_s).astype(jnp.float32)


@jax.jit
def kernel(Xsoft, P):
    del P
    return pl.pallas_call(
        _body,
        grid=(2 * NB,),
        in_specs=[
            pl.BlockSpec((R, BC), lambda t: (0, jnp.minimum(t, NB - 1))),
        ],
        out_specs=pl.BlockSpec(
            (R, BC), lambda t: (0, jnp.where(t < NB, 0, t - NB))
        ),
        out_shape=jax.ShapeDtypeStruct((R, C), jnp.float32),
        scratch_shapes=[
            pltpu.VMEM((R, 1), jnp.float32),
            pltpu.VMEM((R, 1), jnp.int32),
            pltpu.VMEM((R, BC), jnp.int32),
        ],
    )(Xsoft)
